# baseline (speedup 1.0000x reference)
import functools

import jax
import jax.numpy as jnp
from jax import lax
from jax.experimental import pallas as pl
from jax.experimental.pallas import tpu as pltpu
from jax.experimental.pallas import tpu_sc as plsc

EMB = 64
BATCH = 16384
OUT = 2
NW = 32
BPW = BATCH // NW
RPW = 2 * BPW
CHUNK = 128
NCH = RPW // CHUNK
GROUPS = BPW // 16

_mesh = plsc.VectorSubcoreMesh(core_axis_name="c", subcore_axis_name="s")


@functools.partial(
    pl.kernel,
    out_type=jax.ShapeDtypeStruct((BATCH, OUT), jnp.float32),
    mesh=_mesh,
    compiler_params=pltpu.CompilerParams(
        needs_layout_passes=False, use_tc_tiling_on_sc=False),
    scratch_types=[
        pltpu.VMEM((RPW,), jnp.int32),
        pltpu.VMEM((RPW, EMB), jnp.float32),
        pltpu.VMEM((OUT, EMB), jnp.float32),
        pltpu.VMEM((16,), jnp.float32),
        pltpu.VMEM((BPW, OUT), jnp.float32),
        pltpu.SemaphoreType.DMA,
    ],
)
def _srn2vec_sc(xf_hbm, table_hbm, wt_hbm, b_hbm, out_hbm,
                idx_v, rows_v, wt_v, b_v, out_v, sem):
    wid = lax.axis_index("s") * 2 + lax.axis_index("c")
    rbase = wid * RPW

    pltpu.sync_copy(xf_hbm.at[pl.ds(rbase, RPW)], idx_v)
    pltpu.sync_copy(wt_hbm, wt_v)
    pltpu.sync_copy(b_hbm, b_v)

    copies = [
        pltpu.async_copy(
            table_hbm.at[idx_v.at[pl.ds(j * CHUNK, CHUNK)]],
            rows_v.at[pl.ds(j * CHUNK, CHUNK), :],
            sem,
        )
        for j in range(NCH)
    ]
    for c in copies:
        c.wait()

    wt0 = [wt_v[0, pl.ds(k * 16, 16)] for k in range(EMB // 16)]
    wt1 = [wt_v[1, pl.ds(k * 16, 16)] for k in range(EMB // 16)]
    bvec = b_v[...]
    b0 = bvec[0]
    b1 = bvec[1]
    lanes = lax.iota(jnp.int32, 16)
    col0 = jnp.zeros((16,), jnp.int32)
    col1 = col0 + 1

    def group_body(g, carry):
        r0 = g * 32 + 2 * lanes
        r1 = r0 + 1
        acc0 = jnp.zeros((16,), jnp.float32)
        acc1 = jnp.zeros((16,), jnp.float32)
        for d in range(EMB):
            dcol = jnp.full((16,), d, jnp.int32)
            c0 = plsc.load_gather(rows_v, [r0, dcol])
            c1 = plsc.load_gather(rows_v, [r1, dcol])
            p = c0 * c1
            acc0 = acc0 + p * wt0[d // 16][d % 16]
            acc1 = acc1 + p * wt1[d // 16][d % 16]
        y0 = 1.0 / (1.0 + jnp.exp(-(acc0 + b0)))
        y1 = 1.0 / (1.0 + jnp.exp(-(acc1 + b1)))
        bidx = g * 16 + lanes
        plsc.store_scatter(out_v, [bidx, col0], y0)
        plsc.store_scatter(out_v, [bidx, col1], y1)
        return carry

    lax.fori_loop(0, GROUPS, group_body, 0)

    pltpu.sync_copy(out_v, out_hbm.at[pl.ds(wid * BPW, BPW), :])


def kernel(x, table, W_out, b_out):
    xf = x.reshape(-1)
    wt = W_out.T
    bp = jnp.zeros((16,), jnp.float32).at[:OUT].set(b_out)
    return _srn2vec_sc(xf, table, wt, bp)


# speedup vs baseline: 1.4393x; 1.4393x over previous
import functools

import jax
import jax.numpy as jnp
from jax import lax
from jax.experimental import pallas as pl
from jax.experimental.pallas import tpu as pltpu
from jax.experimental.pallas import tpu_sc as plsc

EMB = 64
BATCH = 16384
OUT = 2
NW = 32
BPW = BATCH // NW
GROUPS = BPW // 16
SLOT_R = 9
SLOT_C = 64

_mesh = plsc.VectorSubcoreMesh(core_axis_name="c", subcore_axis_name="s")


def _fire_group(g, xf_hbm, idx_v, tbl_hbm, buf_v, sem):
    iv_a = idx_v[pl.ds(g * 16, 16)]
    iv_b = idx_v[pl.ds(BPW + g * 16, 16)]
    copies = []
    for j in range(16):
        ra = iv_a[j]
        rb = iv_b[j]
        base_a = (ra // 8) * 8
        base_b = (rb // 8) * 8
        copies.append(pltpu.async_copy(
            tbl_hbm.at[pl.ds(base_a, 8), :],
            buf_v.at[pl.ds(j * SLOT_R, 8), :], sem))
        copies.append(pltpu.async_copy(
            tbl_hbm.at[pl.ds(base_b, 8), :],
            buf_v.at[pl.ds((16 + j) * SLOT_R, 8), :], sem))
    return iv_a, iv_b, copies


def _compute_group(g, iv_a, iv_b, buf_v, wt0, wt1, b0, b1, out_v, lanes,
                   col0, col1):
    rows_a = lanes * SLOT_R + lax.rem(iv_a, 8)
    rows_b = (lanes + 16) * SLOT_R + lax.rem(iv_b, 8)
    acc0 = jnp.zeros((16,), jnp.float32)
    acc1 = jnp.zeros((16,), jnp.float32)
    for d in range(EMB):
        dvec = jnp.full((16,), d, jnp.int32)
        c0 = plsc.load_gather(buf_v, [rows_a, dvec])
        c1 = plsc.load_gather(buf_v, [rows_b, dvec])
        p = c0 * c1
        acc0 = acc0 + p * wt0[d // 16][d % 16]
        acc1 = acc1 + p * wt1[d // 16][d % 16]
    y0 = 1.0 / (1.0 + jnp.exp(-(acc0 + b0)))
    y1 = 1.0 / (1.0 + jnp.exp(-(acc1 + b1)))
    bidx = g * 16 + lanes
    plsc.store_scatter(out_v, [bidx, col0], y0)
    plsc.store_scatter(out_v, [bidx, col1], y1)


@functools.partial(
    pl.kernel,
    out_type=jax.ShapeDtypeStruct((BATCH, OUT), jnp.float32),
    mesh=_mesh,
    compiler_params=pltpu.CompilerParams(
        needs_layout_passes=False, use_tc_tiling_on_sc=True),
    scratch_types=[
        pltpu.VMEM((2 * BPW,), jnp.int32),
        pltpu.VMEM((32 * SLOT_R, SLOT_C), jnp.float32),
        pltpu.VMEM((32 * SLOT_R, SLOT_C), jnp.float32),
        pltpu.VMEM((OUT, EMB), jnp.float32),
        pltpu.VMEM((16,), jnp.float32),
        pltpu.VMEM((BPW, OUT), jnp.float32),
        pltpu.SemaphoreType.DMA,
        pltpu.SemaphoreType.DMA,
    ],
)
def _srn2vec_sc(xf_hbm, tbl_hbm, wt_hbm, b_hbm, out_hbm,
                idx_v, buf0_v, buf1_v, wt_v, b_v, out_v, sem0, sem1):
    wid = lax.axis_index("s") * 2 + lax.axis_index("c")
    pltpu.sync_copy(xf_hbm.at[pl.ds(wid * BPW, BPW)], idx_v.at[pl.ds(0, BPW)])
    pltpu.sync_copy(xf_hbm.at[pl.ds(BATCH + wid * BPW, BPW)],
                    idx_v.at[pl.ds(BPW, BPW)])
    pltpu.sync_copy(wt_hbm, wt_v)
    pltpu.sync_copy(b_hbm, b_v)

    wt0 = [wt_v[0, pl.ds(k * 16, 16)] for k in range(EMB // 16)]
    wt1 = [wt_v[1, pl.ds(k * 16, 16)] for k in range(EMB // 16)]
    bvec = b_v[...]
    b0 = bvec[0]
    b1 = bvec[1]
    lanes = lax.iota(jnp.int32, 16)
    col0 = jnp.zeros((16,), jnp.int32)
    col1 = col0 + 1

    def group_loop(g, carry):
        a0, b0v, cp0 = _fire_group(g, xf_hbm, idx_v, tbl_hbm, buf0_v, sem0)
        for c in cp0:
            c.wait()
        _compute_group(g, a0, b0v, buf0_v, wt0, wt1, b0, b1, out_v,
                       lanes, col0, col1)
        return carry

    lax.fori_loop(0, GROUPS, group_loop, 0)

    pltpu.sync_copy(out_v, out_hbm.at[pl.ds(wid * BPW, BPW), :])


def kernel(x, table, W_out, b_out):
    xf = x.T.reshape(-1)
    wt = W_out.T
    bp = jnp.zeros((16,), jnp.float32).at[:OUT].set(b_out)
    return _srn2vec_sc(xf, table, wt, bp)


# speedup vs baseline: 1.4917x; 1.0364x over previous
import functools

import jax
import jax.numpy as jnp
from jax import lax
from jax.experimental import pallas as pl
from jax.experimental.pallas import tpu as pltpu
from jax.experimental.pallas import tpu_sc as plsc

EMB = 64
BATCH = 16384
OUT = 2
NW = 32
BPW = BATCH // NW
GS = 8
NG = BPW // GS
SLOT_R = 9
HALF = 16 * SLOT_R

_mesh = plsc.VectorSubcoreMesh(core_axis_name="c", subcore_axis_name="s")


def _fire_group(g, idx_v, tbl_hbm, buf_v, base, sem):
    iv_a = idx_v[pl.ds(g * GS, 16)]
    iv_b = idx_v[pl.ds(BPW + g * GS, 16)]
    for j in range(GS):
        ra = iv_a[j]
        rb = iv_b[j]
        pltpu.async_copy(
            tbl_hbm.at[pl.ds((ra // 8) * 8, 8), :],
            buf_v.at[pl.ds(base + j * SLOT_R, 8), :], sem)
        pltpu.async_copy(
            tbl_hbm.at[pl.ds((rb // 8) * 8, 8), :],
            buf_v.at[pl.ds(base + (GS + j) * SLOT_R, 8), :], sem)
    return iv_a, iv_b


def _drain_group(tbl_hbm, buf_v, sem):
    pltpu.make_async_copy(
        tbl_hbm.at[pl.ds(0, 128), :], buf_v.at[pl.ds(0, 128), :], sem).wait()


def _compute_group(g, iv_a, iv_b, buf_v, base, wt0, wt1, b0, b1, out_v,
                   lanes, col0, col1, halfmask):
    l8 = lax.rem(lanes, GS)
    rows_a = base + l8 * SLOT_R + lax.rem(iv_a, 8)
    rows_b = base + (GS + l8) * SLOT_R + lax.rem(iv_b, 8)
    acc0 = jnp.zeros((16,), jnp.float32)
    acc1 = jnp.zeros((16,), jnp.float32)
    for d in range(EMB):
        dvec = jnp.full((16,), d, jnp.int32)
        c0 = plsc.load_gather(buf_v, [rows_a, dvec])
        c1 = plsc.load_gather(buf_v, [rows_b, dvec])
        p = c0 * c1
        acc0 = acc0 + p * wt0[d // 16][d % 16]
        acc1 = acc1 + p * wt1[d // 16][d % 16]
    y0 = 1.0 / (1.0 + jnp.exp(-(acc0 + b0)))
    y1 = 1.0 / (1.0 + jnp.exp(-(acc1 + b1)))
    bidx = g * GS + l8
    plsc.store_scatter(out_v, [bidx, col0], y0, mask=halfmask)
    plsc.store_scatter(out_v, [bidx, col1], y1, mask=halfmask)


@functools.partial(
    pl.kernel,
    out_type=jax.ShapeDtypeStruct((BATCH, OUT), jnp.float32),
    mesh=_mesh,
    compiler_params=pltpu.CompilerParams(
        needs_layout_passes=False, use_tc_tiling_on_sc=True),
    scratch_types=[
        pltpu.VMEM((2 * BPW,), jnp.int32),
        pltpu.VMEM((2 * HALF, EMB), jnp.float32),
        pltpu.VMEM((OUT, EMB), jnp.float32),
        pltpu.VMEM((16,), jnp.float32),
        pltpu.VMEM((BPW, OUT), jnp.float32),
        pltpu.SemaphoreType.DMA,
    ],
)
def _srn2vec_sc(xf_hbm, tbl_hbm, wt_hbm, b_hbm, out_hbm,
                idx_v, buf_v, wt_v, b_v, out_v, sem):
    wid = lax.axis_index("s") * 2 + lax.axis_index("c")
    pltpu.sync_copy(xf_hbm.at[pl.ds(wid * BPW, BPW)], idx_v.at[pl.ds(0, BPW)])
    pltpu.sync_copy(xf_hbm.at[pl.ds(BATCH + wid * BPW, BPW)],
                    idx_v.at[pl.ds(BPW, BPW)])
    pltpu.sync_copy(wt_hbm, wt_v)
    pltpu.sync_copy(b_hbm, b_v)

    wt0 = [wt_v[0, pl.ds(k * 16, 16)] for k in range(EMB // 16)]
    wt1 = [wt_v[1, pl.ds(k * 16, 16)] for k in range(EMB // 16)]
    bvec = b_v[...]
    b0 = bvec[0]
    b1 = bvec[1]
    lanes = lax.iota(jnp.int32, 16)
    col0 = jnp.zeros((16,), jnp.int32)
    col1 = col0 + 1
    halfmask = lanes < GS

    _fire_group(0, idx_v, tbl_hbm, buf_v, 0, sem)

    def group_loop(g, carry):
        nbase = lax.rem(g + 1, 2) * HALF
        base = lax.rem(g, 2) * HALF

        @pl.when(g < NG - 1)
        def _():
            _fire_group(g + 1, idx_v, tbl_hbm, buf_v, nbase, sem)

        iv_a = idx_v[pl.ds(g * GS, 16)]
        iv_b = idx_v[pl.ds(BPW + g * GS, 16)]
        _drain_group(tbl_hbm, buf_v, sem)
        _compute_group(g, iv_a, iv_b, buf_v, base, wt0, wt1, b0, b1, out_v,
                       lanes, col0, col1, halfmask)
        return carry

    lax.fori_loop(0, NG, group_loop, 0)

    pltpu.sync_copy(out_v, out_hbm.at[pl.ds(wid * BPW, BPW), :])


def kernel(x, table, W_out, b_out):
    xf = x.T.reshape(-1)
    wt = W_out.T
    bp = jnp.zeros((16,), jnp.float32).at[:OUT].set(b_out)
    return _srn2vec_sc(xf, table, wt, bp)
